# Initial kernel scaffold; baseline (speedup 1.0000x reference)
import jax
import jax.numpy as jnp
from jax.experimental import pallas as pl


def kernel(user_embs_raw, tags_embedding_table):
    raise NotImplementedError("write your pallas kernel here")


# speedup vs baseline: 1.5466x; 1.5466x over previous
import functools

import jax
import jax.numpy as jnp
from jax import lax
from jax.experimental import pallas as pl

TOP_K = 20
CARRY_W = 32
CHUNK = 2048
BU = 128
NEG_INF = float("-inf")
BIG_I32 = 2**31 - 1


def _topk_kernel(u_ref, t_ref, out_ref, *, n_tags, n_tags_pad):
    n_chunks = n_tags_pad // CHUNK
    u = u_ref[...]

    lane_c = lax.broadcasted_iota(jnp.int32, (BU, CARRY_W), 1)
    init_vals = jnp.full((BU, CARRY_W), NEG_INF, dtype=jnp.float32)
    init_idx = -(lane_c + 1)

    def chunk_body(c, carry):
        cvals, cidx = carry
        t_chunk = t_ref[pl.ds(c * CHUNK, CHUNK), :]
        scores = lax.dot_general(
            u, t_chunk,
            dimension_numbers=(((1,), (1,)), ((), ())),
            preferred_element_type=jnp.float32,
        )
        gidx = c * CHUNK + lax.broadcasted_iota(jnp.int32, (BU, CHUNK), 1)
        scores = jnp.where(gidx < n_tags, scores, NEG_INF)

        work = jnp.concatenate([cvals, scores], axis=1)
        wids = jnp.concatenate([cidx, gidx], axis=1)

        nvals = init_vals
        nidx = init_idx
        for k in range(TOP_K):
            m = jnp.max(work, axis=1, keepdims=True)
            cand = jnp.where(work == m, wids, BIG_I32)
            sel = jnp.min(cand, axis=1, keepdims=True)
            nvals = jnp.where(lane_c == k, m, nvals)
            nidx = jnp.where(lane_c == k, sel, nidx)
            work = jnp.where(wids == sel, NEG_INF, work)
        return nvals, nidx

    _, fidx = lax.fori_loop(0, n_chunks, chunk_body, (init_vals, init_idx))
    out_ref[...] = fidx


def kernel(user_embs_raw, tags_embedding_table):
    n_users, dim = user_embs_raw.shape
    n_tags = tags_embedding_table.shape[0]
    n_tags_pad = ((n_tags + CHUNK - 1) // CHUNK) * CHUNK
    tags_pad = jnp.pad(tags_embedding_table, ((0, n_tags_pad - n_tags), (0, 0)))

    grid = (n_users // BU,)
    out = pl.pallas_call(
        functools.partial(_topk_kernel, n_tags=n_tags, n_tags_pad=n_tags_pad),
        grid=grid,
        in_specs=[
            pl.BlockSpec((BU, dim), lambda i: (i, 0)),
            pl.BlockSpec((n_tags_pad, dim), lambda i: (0, 0)),
        ],
        out_specs=pl.BlockSpec((BU, CARRY_W), lambda i: (i, 0)),
        out_shape=jax.ShapeDtypeStruct((n_users, CARRY_W), jnp.int32),
    )(user_embs_raw, tags_pad)
    return out[:, :TOP_K]


# speedup vs baseline: 3.5402x; 2.2891x over previous
import functools

import jax
import jax.numpy as jnp
from jax import lax
from jax.experimental import pallas as pl
from jax.experimental.pallas import tpu as pltpu
from jax.experimental.pallas import tpu_sc as plsc

TOP_K = 20
GROUP = 128
CHUNK = 8192
BU = 64
BUF = 128
GLIST = 48
OUTW = 32
NEG_INF = float("-inf")
BIG_I32 = 2**31 - 1


def _score_kernel(u_ref, t_ref, s_ref, a_ref, msc, *, n_tags, n_chunks):
    mscw = n_chunks * 128
    gpc = CHUNK // GROUP
    c = pl.program_id(1)
    u = u_ref[...]
    t_chunk = t_ref[...]
    scores = lax.dot_general(
        u, t_chunk,
        dimension_numbers=(((1,), (1,)), ((), ())),
        preferred_element_type=jnp.float32,
    )
    gidx = c * CHUNK + lax.broadcasted_iota(jnp.int32, (BU, CHUNK), 1)
    scores = jnp.where(gidx < n_tags, scores, NEG_INF)
    s_ref[...] = scores

    gm = jnp.max(scores.reshape(BU, gpc, GROUP), axis=-1)
    pad = jnp.full((BU, 128 - gpc), NEG_INF, jnp.float32)
    msc[:, pl.ds(c * 128, 128)] = jnp.concatenate([gm, pad], axis=1)

    @pl.when(c == n_chunks - 1)
    def _():
        morig = msc[...]
        gix = lax.broadcasted_iota(jnp.int32, (BU, mscw), 1)
        lanes = lax.broadcasted_iota(jnp.int32, (BU, GLIST), 1)

        def ext_body(k, carry):
            work, acc = carry
            mx = jnp.max(work, axis=1, keepdims=True)
            sel = jnp.min(jnp.where(work == mx, gix, BIG_I32), axis=1,
                          keepdims=True)
            acc = jnp.where(lanes == k, sel, acc)
            work = jnp.where(gix == sel, NEG_INF, work)
            return work, acc

        _, ids = lax.fori_loop(
            0, GLIST, ext_body, (morig, jnp.zeros((BU, GLIST), jnp.int32)))
        a_ref[...] = (ids >> 7) * gpc + (ids & 127)


def _gather_kernel(s2_hbm, a_hbm, out_hbm, auxv, bidv, gbuf, sem, *,
                   n_groups, rows_per_worker):
    wid = lax.axis_index("s") * 2 + lax.axis_index("c")

    def row_body(r, _carry):
        u = wid * rows_per_worker + r
        pltpu.sync_copy(a_hbm.at[u], auxv)
        for kk in range(GLIST // 16):
            bidv[pl.ds(kk * 16, 16)] = (auxv[pl.ds(kk * 16, 16)]
                                        + u * n_groups)
        pltpu.async_copy(s2_hbm.at[bidv], gbuf, sem).wait()
        pltpu.sync_copy(gbuf, out_hbm.at[u])
        return _carry

    lax.fori_loop(0, rows_per_worker, row_body, 0)


def _final_kernel(c_ref, a_ref, o_ref):
    w = GLIST * GROUP
    vals = c_ref[...]
    gl = a_ref[...]
    base = gl.reshape(BUF, GLIST, 1) * GROUP
    lane = lax.broadcasted_iota(jnp.int32, (BUF, GLIST, GROUP), 2)
    wids = (base + lane).reshape(BUF, w)
    lane32 = lax.broadcasted_iota(jnp.int32, (BUF, OUTW), 1)

    work = vals
    out = jnp.zeros((BUF, OUTW), jnp.int32)
    for k in range(TOP_K):
        mx = jnp.max(work, axis=1, keepdims=True)
        sel = jnp.min(jnp.where(work == mx, wids, BIG_I32), axis=1,
                      keepdims=True)
        out = jnp.where(lane32 == k, sel, out)
        work = jnp.where(wids == sel, NEG_INF, work)
    o_ref[...] = out


def kernel(user_embs_raw, tags_embedding_table):
    n_users, dim = user_embs_raw.shape
    n_tags = tags_embedding_table.shape[0]
    n_chunks = (n_tags + CHUNK - 1) // CHUNK
    n_tags_pad = n_chunks * CHUNK
    n_groups = n_tags_pad // GROUP
    tags_pad = jnp.pad(tags_embedding_table, ((0, n_tags_pad - n_tags), (0, 0)))

    scores, aux = pl.pallas_call(
        functools.partial(_score_kernel, n_tags=n_tags, n_chunks=n_chunks),
        grid=(n_users // BU, n_chunks),
        in_specs=[
            pl.BlockSpec((BU, dim), lambda i, c: (i, 0)),
            pl.BlockSpec((CHUNK, dim), lambda i, c: (c, 0)),
        ],
        out_specs=[
            pl.BlockSpec((BU, CHUNK), lambda i, c: (i, c)),
            pl.BlockSpec((BU, GLIST), lambda i, c: (i, 0)),
        ],
        out_shape=[
            jax.ShapeDtypeStruct((n_users, n_tags_pad), jnp.float32),
            jax.ShapeDtypeStruct((n_users, GLIST), jnp.int32),
        ],
        scratch_shapes=[pltpu.VMEM((BU, n_chunks * 128), jnp.float32)],
    )(user_embs_raw, tags_pad)

    s2 = scores.reshape(n_users * n_groups, GROUP)
    rows_per_worker = n_users // 32

    mesh = plsc.VectorSubcoreMesh(core_axis_name="c", subcore_axis_name="s")
    compact = pl.kernel(
        functools.partial(_gather_kernel, n_groups=n_groups,
                          rows_per_worker=rows_per_worker),
        mesh=mesh,
        out_type=jax.ShapeDtypeStruct((n_users, GLIST, GROUP), jnp.float32),
        scratch_types=[
            pltpu.VMEM((GLIST,), jnp.int32),
            pltpu.VMEM((GLIST,), jnp.int32),
            pltpu.VMEM((GLIST, GROUP), jnp.float32),
            pltpu.SemaphoreType.DMA,
        ],
    )(s2, aux)

    out = pl.pallas_call(
        _final_kernel,
        grid=(n_users // BUF,),
        in_specs=[
            pl.BlockSpec((BUF, GLIST * GROUP), lambda i: (i, 0)),
            pl.BlockSpec((BUF, GLIST), lambda i: (i, 0)),
        ],
        out_specs=pl.BlockSpec((BUF, OUTW), lambda i: (i, 0)),
        out_shape=jax.ShapeDtypeStruct((n_users, OUTW), jnp.int32),
    )(compact.reshape(n_users, GLIST * GROUP), aux)
    return out[:, :TOP_K]


# speedup vs baseline: 4.2820x; 1.2095x over previous
import functools

import jax
import jax.numpy as jnp
from jax import lax
from jax.experimental import pallas as pl
from jax.experimental.pallas import tpu as pltpu
from jax.experimental.pallas import tpu_sc as plsc

TOP_K = 20
GROUP = 128
CHUNK = 8192
BU = 128
BUF = 128
GLIST = 48
OUTW = 32
NEG_INF = float("-inf")
BIG_I32 = 2**31 - 1


def _score_kernel(u_ref, t_ref, s_ref, a_ref, msc, *, n_tags, n_chunks):
    mscw = n_chunks * 128
    gpc = CHUNK // GROUP
    c = pl.program_id(1)
    u = u_ref[...]
    t_chunk = t_ref[...]
    scores = lax.dot_general(
        u, t_chunk,
        dimension_numbers=(((1,), (1,)), ((), ())),
        preferred_element_type=jnp.float32,
    )
    gidx = c * CHUNK + lax.broadcasted_iota(jnp.int32, (BU, CHUNK), 1)
    scores = jnp.where(gidx < n_tags, scores, NEG_INF)
    s_ref[...] = scores

    gm = jnp.max(scores.reshape(BU, gpc, GROUP), axis=-1)
    pad = jnp.full((BU, 128 - gpc), NEG_INF, jnp.float32)
    msc[:, pl.ds(c * 128, 128)] = jnp.concatenate([gm, pad], axis=1)

    @pl.when(c == n_chunks - 1)
    def _():
        morig = msc[...]
        gix = lax.broadcasted_iota(jnp.int32, (BU, mscw), 1)
        lanes = lax.broadcasted_iota(jnp.int32, (BU, GLIST), 1)

        def ext_body(k, carry):
            work, acc = carry
            mx = jnp.max(work, axis=1, keepdims=True)
            sel = jnp.min(jnp.where(work == mx, gix, BIG_I32), axis=1,
                          keepdims=True)
            acc = jnp.where(lanes == k, sel, acc)
            work = jnp.where(gix == sel, NEG_INF, work)
            return work, acc

        _, ids = lax.fori_loop(
            0, GLIST, ext_body, (morig, jnp.zeros((BU, GLIST), jnp.int32)))
        a_ref[...] = (ids >> 7) * gpc + (ids & 127)


def _gather_kernel(s2_hbm, a_hbm, out_hbm, auxv, bidv, gbuf, sem, *,
                   n_groups, rows_per_worker):
    wid = lax.axis_index("s") * 2 + lax.axis_index("c")

    def row_body(r, _carry):
        u = wid * rows_per_worker + r
        pltpu.sync_copy(a_hbm.at[u], auxv)
        for kk in range(GLIST // 16):
            bidv[pl.ds(kk * 16, 16)] = (auxv[pl.ds(kk * 16, 16)]
                                        + u * n_groups)
        pltpu.async_copy(s2_hbm.at[bidv], gbuf, sem).wait()
        pltpu.sync_copy(gbuf, out_hbm.at[u])
        return _carry

    lax.fori_loop(0, rows_per_worker, row_body, 0)


def _final_kernel(c_ref, a_ref, o_ref):
    w = GLIST * GROUP
    vals = c_ref[...]
    gl = a_ref[...]
    base = gl.reshape(BUF, GLIST, 1) * GROUP
    lane = lax.broadcasted_iota(jnp.int32, (BUF, GLIST, GROUP), 2)
    wids = (base + lane).reshape(BUF, w)
    lane32 = lax.broadcasted_iota(jnp.int32, (BUF, OUTW), 1)

    work = vals
    out = jnp.zeros((BUF, OUTW), jnp.int32)
    for k in range(TOP_K):
        mx = jnp.max(work, axis=1, keepdims=True)
        sel = jnp.min(jnp.where(work == mx, wids, BIG_I32), axis=1,
                      keepdims=True)
        out = jnp.where(lane32 == k, sel, out)
        work = jnp.where(wids == sel, NEG_INF, work)
    o_ref[...] = out


def kernel(user_embs_raw, tags_embedding_table):
    n_users, dim = user_embs_raw.shape
    n_tags = tags_embedding_table.shape[0]
    n_chunks = (n_tags + CHUNK - 1) // CHUNK
    n_tags_pad = n_chunks * CHUNK
    n_groups = n_tags_pad // GROUP
    tags_pad = jnp.pad(tags_embedding_table, ((0, n_tags_pad - n_tags), (0, 0)))

    scores, aux = pl.pallas_call(
        functools.partial(_score_kernel, n_tags=n_tags, n_chunks=n_chunks),
        grid=(n_users // BU, n_chunks),
        in_specs=[
            pl.BlockSpec((BU, dim), lambda i, c: (i, 0)),
            pl.BlockSpec((CHUNK, dim), lambda i, c: (c, 0)),
        ],
        out_specs=[
            pl.BlockSpec((BU, CHUNK), lambda i, c: (i, c)),
            pl.BlockSpec((BU, GLIST), lambda i, c: (i, 0)),
        ],
        out_shape=[
            jax.ShapeDtypeStruct((n_users, n_tags_pad), jnp.float32),
            jax.ShapeDtypeStruct((n_users, GLIST), jnp.int32),
        ],
        scratch_shapes=[pltpu.VMEM((BU, n_chunks * 128), jnp.float32)],
    )(user_embs_raw, tags_pad)

    s2 = scores.reshape(n_users * n_groups, GROUP)
    rows_per_worker = n_users // 32

    mesh = plsc.VectorSubcoreMesh(core_axis_name="c", subcore_axis_name="s")
    compact = pl.kernel(
        functools.partial(_gather_kernel, n_groups=n_groups,
                          rows_per_worker=rows_per_worker),
        mesh=mesh,
        out_type=jax.ShapeDtypeStruct((n_users, GLIST, GROUP), jnp.float32),
        scratch_types=[
            pltpu.VMEM((GLIST,), jnp.int32),
            pltpu.VMEM((GLIST,), jnp.int32),
            pltpu.VMEM((GLIST, GROUP), jnp.float32),
            pltpu.SemaphoreType.DMA,
        ],
    )(s2, aux)

    out = pl.pallas_call(
        _final_kernel,
        grid=(n_users // BUF,),
        in_specs=[
            pl.BlockSpec((BUF, GLIST * GROUP), lambda i: (i, 0)),
            pl.BlockSpec((BUF, GLIST), lambda i: (i, 0)),
        ],
        out_specs=pl.BlockSpec((BUF, OUTW), lambda i: (i, 0)),
        out_shape=jax.ShapeDtypeStruct((n_users, OUTW), jnp.int32),
    )(compact.reshape(n_users, GLIST * GROUP), aux)
    return out[:, :TOP_K]
